# baseline (speedup 1.0000x reference)
import functools

import jax
import jax.numpy as jnp
from jax import lax
from jax.experimental import pallas as pl
from jax.experimental.pallas import tpu as pltpu
from jax.experimental.pallas import tpu_sc as plsc


@functools.lru_cache(maxsize=None)
def _build(V, S_s, S_m, D, R, cs, mrep):
    info = plsc.get_sparse_core_info()
    nc, ns = info.num_cores, info.num_subcores
    nw = nc * ns
    rpw = R // nw
    n_chunks = rpw // cs
    mesh = plsc.VectorSubcoreMesh(core_axis_name="c", subcore_axis_name="s")

    @functools.partial(
        pl.kernel,
        mesh=mesh,
        out_type=jax.ShapeDtypeStruct((R, S_m + S_s, D), jnp.float32),
        scratch_types=[
            pltpu.VMEM((n_chunks, cs), jnp.int32),
            pltpu.VMEM((cs, S_s, D), jnp.float32),
            pltpu.VMEM((cs, S_s, D), jnp.float32),
            pltpu.VMEM((cs, S_s, D), jnp.float32),
            pltpu.VMEM((mrep, S_m, D), jnp.float32),
            pltpu.SemaphoreType.DMA,
            pltpu.SemaphoreType.DMA,
            pltpu.SemaphoreType.DMA,
            pltpu.SemaphoreType.DMA,
            pltpu.SemaphoreType.DMA,
            pltpu.SemaphoreType.DMA,
            pltpu.SemaphoreType.DMA,
            pltpu.SemaphoreType.DMA,
        ],
    )
    def k(market_hbm, ids2d_hbm, table_hbm, out_hbm,
          idx_v, g0, g1, g2, mk_v, gs0, gs1, gs2, ws0, ws1, ws2, msem, isem):
        wid = lax.axis_index("s") * nc + lax.axis_index("c")
        base = wid * rpw
        gath_v = (g0, g1, g2)
        gsem = (gs0, gs1, gs2)
        wsem = (ws0, ws1, ws2)

        ids_cp = pltpu.async_copy(
            ids2d_hbm.at[pl.ds(wid * n_chunks, n_chunks), :], idx_v, isem)
        mfill = [pltpu.async_copy(market_hbm, mk_v.at[i], msem)
                 for i in range(mrep)]
        for d in mfill:
            d.wait()
        mw = [pltpu.async_copy(
                  mk_v,
                  out_hbm.at[pl.ds(base + j * mrep, mrep), pl.ds(0, S_m), :],
                  msem)
              for j in range(rpw // mrep)]
        ids_cp.wait()

        def start_gather(c, b):
            return pltpu.async_copy(table_hbm.at[idx_v.at[c]], gath_v[b],
                                    gsem[b])

        g = [start_gather(0, 0), start_gather(1, 1), None]
        w = [None, None, None]
        for c in range(n_chunks):
            b = c % 3
            pb = (c + 2) % 3
            if c + 2 < n_chunks:
                if w[pb] is not None:
                    w[pb].wait()
                g[pb] = start_gather(c + 2, pb)
            row0 = base + c * cs
            g[b].wait()
            w[b] = pltpu.async_copy(
                gath_v[b], out_hbm.at[pl.ds(row0, cs), pl.ds(S_m, S_s), :],
                wsem[b])
        for b in range(3):
            if w[b] is not None:
                w[b].wait()
        for d in mw:
            d.wait()

    return k


def kernel(market_memory, symbol_memory, symbol_ids, batch_size, num_symbols):
    S_m, D = market_memory.shape
    V, S_s, _ = symbol_memory.shape
    b, n = symbol_ids.shape
    R = b * n
    cs = 16
    ids2d = symbol_ids.reshape(R // cs, cs).astype(jnp.int32)
    k = _build(V, S_s, S_m, D, R, cs, 16)
    out = k(market_memory, ids2d, symbol_memory)
    return out.reshape(b, n, S_m + S_s, D)


# speedup vs baseline: 1.1162x; 1.1162x over previous
import functools

import jax
import jax.numpy as jnp
from jax import lax
from jax.experimental import pallas as pl
from jax.experimental.pallas import tpu as pltpu
from jax.experimental.pallas import tpu_sc as plsc


@functools.lru_cache(maxsize=None)
def _build(V, S_s, S_m, D, R, cs, mrep):
    info = plsc.get_sparse_core_info()
    nc, ns = info.num_cores, info.num_subcores
    nw = nc * ns
    rpw = R // nw
    n_chunks = rpw // cs
    mesh = plsc.VectorSubcoreMesh(core_axis_name="c", subcore_axis_name="s")

    @functools.partial(
        pl.kernel,
        mesh=mesh,
        out_type=jax.ShapeDtypeStruct((R, S_m + S_s, D), jnp.float32),
        scratch_types=[
            pltpu.VMEM((n_chunks, cs), jnp.int32),
            pltpu.VMEM((cs, S_s, D), jnp.float32),
            pltpu.VMEM((cs, S_s, D), jnp.float32),
            pltpu.VMEM((cs, S_s, D), jnp.float32),
            pltpu.VMEM((mrep, S_m, D), jnp.float32),
            pltpu.SemaphoreType.DMA,
            pltpu.SemaphoreType.DMA,
            pltpu.SemaphoreType.DMA,
            pltpu.SemaphoreType.DMA,
            pltpu.SemaphoreType.DMA,
            pltpu.SemaphoreType.DMA,
            pltpu.SemaphoreType.DMA,
            pltpu.SemaphoreType.DMA,
        ],
    )
    def k(market_hbm, ids2d_hbm, table_hbm, out_hbm,
          idx_v, g0, g1, g2, mk_v, gs0, gs1, gs2, ws0, ws1, ws2, msem, isem):
        wid = lax.axis_index("s") * nc + lax.axis_index("c")
        base = wid * rpw
        gath_v = (g0, g1, g2)
        gsem = (gs0, gs1, gs2)
        wsem = (ws0, ws1, ws2)
        mpc = cs // mrep

        def market_writes(c):
            row0 = base + c * cs
            return [pltpu.async_copy(
                        mk_v,
                        out_hbm.at[pl.ds(row0 + j * mrep, mrep),
                                   pl.ds(0, S_m), :],
                        msem)
                    for j in range(mpc)]

        ids_cp = pltpu.async_copy(
            ids2d_hbm.at[pl.ds(wid * n_chunks, n_chunks), :], idx_v, isem)
        mfill = [pltpu.async_copy(market_hbm, mk_v.at[i], msem)
                 for i in range(mrep)]
        for d in mfill:
            d.wait()
        ids_cp.wait()

        def start_gather(c, b):
            return pltpu.async_copy(table_hbm.at[idx_v.at[c]], gath_v[b],
                                    gsem[b])

        g = [start_gather(0, 0), start_gather(1, 1), None]
        w = [None, None, None]
        mq = [market_writes(0), market_writes(1)]
        for c in range(n_chunks):
            b = c % 3
            pb = (c + 2) % 3
            if c + 2 < n_chunks:
                if w[pb] is not None:
                    w[pb].wait()
                g[pb] = start_gather(c + 2, pb)
                mq.append(market_writes(c + 2))
            if len(mq) > 3:
                for d in mq.pop(0):
                    d.wait()
            row0 = base + c * cs
            g[b].wait()
            w[b] = pltpu.async_copy(
                gath_v[b], out_hbm.at[pl.ds(row0, cs), pl.ds(S_m, S_s), :],
                wsem[b])
        for b in range(3):
            if w[b] is not None:
                w[b].wait()
        for mws in mq:
            for d in mws:
                d.wait()

    return k


def kernel(market_memory, symbol_memory, symbol_ids, batch_size, num_symbols):
    S_m, D = market_memory.shape
    V, S_s, _ = symbol_memory.shape
    b, n = symbol_ids.shape
    R = b * n
    cs = 32
    ids2d = symbol_ids.reshape(R // cs, cs).astype(jnp.int32)
    k = _build(V, S_s, S_m, D, R, cs, 8)
    out = k(market_memory, ids2d, symbol_memory)
    return out.reshape(b, n, S_m + S_s, D)


# speedup vs baseline: 1.1739x; 1.0516x over previous
import functools

import jax
import jax.numpy as jnp
from jax import lax
from jax.experimental import pallas as pl
from jax.experimental.pallas import tpu as pltpu
from jax.experimental.pallas import tpu_sc as plsc


@functools.lru_cache(maxsize=None)
def _build(V, S_s, S_m, D, R, cs, mrep):
    info = plsc.get_sparse_core_info()
    nc, ns = info.num_cores, info.num_subcores
    nw = nc * ns
    rpw = R // nw
    n_chunks = rpw // cs
    mesh = plsc.VectorSubcoreMesh(core_axis_name="c", subcore_axis_name="s")

    @functools.partial(
        pl.kernel,
        mesh=mesh,
        out_type=jax.ShapeDtypeStruct((R, S_m + S_s, D), jnp.float32),
        scratch_types=[
            pltpu.VMEM((n_chunks, cs), jnp.int32),
            pltpu.VMEM((cs, S_s, D), jnp.float32),
            pltpu.VMEM((cs, S_s, D), jnp.float32),
            pltpu.VMEM((cs, S_s, D), jnp.float32),
            pltpu.VMEM((mrep, S_m, D), jnp.float32),
            pltpu.SemaphoreType.DMA,
            pltpu.SemaphoreType.DMA,
            pltpu.SemaphoreType.DMA,
            pltpu.SemaphoreType.DMA,
            pltpu.SemaphoreType.DMA,
            pltpu.SemaphoreType.DMA,
            pltpu.SemaphoreType.DMA,
        ],
    )
    def k(market_hbm, ids2d_hbm, table_hbm, out_hbm,
          idx_v, g0, g1, g2, mk_v, gs0, gs1, gs2, ws0, ws1, ws2, msem):
        wid = lax.axis_index("s") * nc + lax.axis_index("c")
        base = wid * rpw
        gath_v = (g0, g1, g2)
        gsem = (gs0, gs1, gs2)
        wsem = (ws0, ws1, ws2)

        pltpu.sync_copy(ids2d_hbm.at[pl.ds(wid * n_chunks, n_chunks), :], idx_v)
        mfill = [pltpu.async_copy(market_hbm, mk_v.at[i], msem)
                 for i in range(mrep)]

        def start_gather(c, b):
            return pltpu.async_copy(table_hbm.at[idx_v.at[c]], gath_v[b],
                                    gsem[b])

        g = [start_gather(0, 0), start_gather(1, 1), None]
        w = [None, None, None]
        m = [[], [], []]
        for d in mfill:
            d.wait()
        for c in range(n_chunks):
            b = c % 3
            pb = (c + 2) % 3
            if c + 2 < n_chunks:
                if w[pb] is not None:
                    w[pb].wait()
                g[pb] = start_gather(c + 2, pb)
            row0 = base + c * cs
            for d in m[b]:
                d.wait()
            m[b] = []
            g[b].wait()
            w[b] = pltpu.async_copy(
                gath_v[b], out_hbm.at[pl.ds(row0, cs), pl.ds(S_m, S_s), :],
                wsem[b])
            for j in range(cs // mrep):
                m[b].append(pltpu.async_copy(
                    mk_v,
                    out_hbm.at[pl.ds(row0 + j * mrep, mrep), pl.ds(0, S_m), :],
                    msem))
        for b in range(3):
            if w[b] is not None:
                w[b].wait()
            for d in m[b]:
                d.wait()

    return k


def kernel(market_memory, symbol_memory, symbol_ids, batch_size, num_symbols):
    S_m, D = market_memory.shape
    V, S_s, _ = symbol_memory.shape
    b, n = symbol_ids.shape
    R = b * n
    cs = 32
    ids2d = symbol_ids.reshape(R // cs, cs).astype(jnp.int32)
    k = _build(V, S_s, S_m, D, R, cs, 8)
    out = k(market_memory, ids2d, symbol_memory)
    return out.reshape(b, n, S_m + S_s, D)
